# baseline (speedup 1.0000x reference)
import functools

import jax
import jax.numpy as jnp
from jax import lax
from jax.experimental import pallas as pl
from jax.experimental.pallas import tpu as pltpu
from jax.experimental.pallas import tpu_sc as plsc

_IOU_THRESHOLD = 0.6
_K = 50
_LANES = 128
_RB = 8
_NPAD = 5120
_ROWS = _NPAD // _LANES
_NBLK = _ROWS // _RB
_NSETS = 4
_NCHUNK = 8
_CHUNK = _NPAD // _NCHUNK
_NV = _CHUNK // 16
_NP = 24


def _clip_dt(pu, pv, du, dv, rxu, rxv, hx, hy):
    ta = -(hx + pu) * rxu
    tb = (hx - pu) * rxu
    lo_u = jnp.minimum(ta, tb)
    hi_u = jnp.maximum(ta, tb)
    tc = -(hy + pv) * rxv
    td = (hy - pv) * rxv
    lo_v = jnp.minimum(tc, td)
    hi_v = jnp.maximum(tc, td)
    t0 = jnp.maximum(jnp.maximum(lo_u, lo_v), 0.0)
    t1 = jnp.minimum(jnp.minimum(hi_u, hi_v), 1.0)
    return jnp.maximum(t1 - t0, 0.0)


def _pair_step(p, g, k, best, bki):
    (pax0, pax1, pax2, pax3, pay0, pay1, pay2, pay3,
     e0x, e0y, e1x, e1y, x, y, ca, sa, hxa, hya, za1, za2, va) = p
    (cx, cy, cb, sb, hx, hy, zb1, zb2, vb,
     qx0, qy0, qx1, qy1, qx2, qy2, qx3, qy3, d0x, d0y, d1x, d1y) = g

    us = []
    vs = []
    for pax, pay in ((pax0, pay0), (pax1, pay1), (pax2, pay2), (pax3, pay3)):
        tx = pax - cx
        ty = pay - cy
        us.append(cb * tx + sb * ty)
        vs.append(cb * ty - sb * tx)
    e0u = cb * e0x + sb * e0y
    e0v = cb * e0y - sb * e0x
    e1u = cb * e1x + sb * e1y
    e1v = cb * e1y - sb * e1x
    r0u = 1.0 / e0u
    r0v = 1.0 / e0v
    r1u = 1.0 / e1u
    r1v = 1.0 / e1v
    eds = ((e0u, e0v, r0u, r0v), (e1u, e1v, r1u, r1v),
           (-e0u, -e0v, -r0u, -r0v), (-e1u, -e1v, -r1u, -r1v))
    area2 = jnp.zeros_like(x)
    for j in range(4):
        du, dv, ru, rv = eds[j]
        dt = _clip_dt(us[j], vs[j], du, dv, ru, rv, hx, hy)
        area2 = area2 + dt * (us[j] * dv - vs[j] * du)

    qs = []
    for qxj, qyj in ((qx0, qy0), (qx1, qy1), (qx2, qy2), (qx3, qy3)):
        tx = qxj - x
        ty = qyj - y
        qs.append((ca * tx + sa * ty, ca * ty - sa * tx))
    d0u = ca * d0x + sa * d0y
    d0v = ca * d0y - sa * d0x
    d1u = ca * d1x + sa * d1y
    d1v = ca * d1y - sa * d1x
    s0u = 1.0 / d0u
    s0v = 1.0 / d0v
    s1u = 1.0 / d1u
    s1v = 1.0 / d1v
    bds = ((d0u, d0v, s0u, s0v), (d1u, d1v, s1u, s1v),
           (-d0u, -d0v, -s0u, -s0v), (-d1u, -d1v, -s1u, -s1v))
    dtsum = jnp.zeros_like(x)
    for j in range(4):
        du, dv, ru, rv = bds[j]
        dtsum = dtsum + _clip_dt(qs[j][0], qs[j][1], du, dv, ru, rv, hxa, hya)
    area2 = area2 + dtsum * (2.0 * hx * hy)

    area = jnp.maximum(0.5 * area2, 0.0)
    h = jnp.maximum(jnp.minimum(za2, zb2) - jnp.maximum(za1, zb1), 0.0)
    inter = area * h
    iou = inter / jnp.maximum(va + vb - inter, 1e-6)
    upd = iou > best
    best = jnp.where(upd, iou, best)
    bki = jnp.where(upd, jnp.broadcast_to(k, bki.shape), bki)
    return best, bki


def _prep_body(pred_ref, out_ref):
    x = pred_ref[0, 0]
    y = pred_ref[0, 1]
    z = pred_ref[0, 2]
    dxa = pred_ref[0, 3]
    dya = pred_ref[0, 4]
    dza = pred_ref[0, 5]
    r = pred_ref[0, 6]
    ca = jnp.cos(r)
    sa = jnp.sin(r)
    hxa = 0.5 * dxa
    hya = 0.5 * dya
    cxh = ca * hxa
    sxh = sa * hxa
    cyh = ca * hya
    syh = sa * hya
    out_ref[0, 0] = x + cxh - syh
    out_ref[0, 1] = x - cxh - syh
    out_ref[0, 2] = x - cxh + syh
    out_ref[0, 3] = x + cxh + syh
    out_ref[0, 4] = y + sxh + cyh
    out_ref[0, 5] = y - sxh + cyh
    out_ref[0, 6] = y - sxh - cyh
    out_ref[0, 7] = y + sxh - cyh
    out_ref[0, 8] = -2.0 * cxh
    out_ref[0, 9] = -2.0 * sxh
    out_ref[0, 10] = 2.0 * syh
    out_ref[0, 11] = -2.0 * cyh
    out_ref[0, 12] = x
    out_ref[0, 13] = y
    out_ref[0, 14] = ca
    out_ref[0, 15] = sa
    out_ref[0, 16] = hxa
    out_ref[0, 17] = hya
    out_ref[0, 18] = z - 0.5 * dza
    out_ref[0, 19] = z + 0.5 * dza
    out_ref[0, 20] = dxa * dya * dza
    zero = jnp.zeros_like(x)
    out_ref[0, 21] = zero
    out_ref[0, 22] = zero
    out_ref[0, 23] = zero


def _sc_assign_body(planes_hbm, gt_hbm, out_hbm, pv, gv, ov):
    cid = lax.axis_index("c")
    sid = lax.axis_index("s")
    wid = sid * 2 + cid
    st = wid // _NCHUNK
    ck = lax.rem(wid, _NCHUNK)
    pltpu.sync_copy(planes_hbm.at[st, ck], pv)
    pltpu.sync_copy(gt_hbm.at[st], gv)

    def outer(v, carry_o):
        off = v * 16
        p = tuple(pv[pl.ds(j * _CHUNK + off, 16)] for j in range(21))

        def inner(k, carry):
            best, bki = carry
            g = tuple(gv[pl.ds(k * 384 + j * 16, 16)] for j in range(21))
            return _pair_step(p, g, k, best, bki)

        best, bki = lax.fori_loop(
            0, _K, inner,
            (jnp.full((16,), -1.0, jnp.float32), jnp.zeros((16,), jnp.int32)))
        ov[pl.ds(off, 16)] = jnp.where(best < _IOU_THRESHOLD,
                                       jnp.full((16,), -1, jnp.int32), bki)
        return carry_o

    lax.fori_loop(0, _NV, outer, 0)
    pltpu.sync_copy(ov, out_hbm.at[st, ck])


def _gt_scalars(gt):
    cx = gt[..., 0]
    cy = gt[..., 1]
    zc = gt[..., 2]
    dx = gt[..., 3]
    dy = gt[..., 4]
    dz = gt[..., 5]
    rr = gt[..., 6]
    cb = jnp.cos(rr)
    sb = jnp.sin(rr)
    hx = 0.5 * dx
    hy = 0.5 * dy
    cxh = cb * hx
    sxh = sb * hx
    cyh = cb * hy
    syh = sb * hy
    qx0 = cx + cxh - syh
    qy0 = cy + sxh + cyh
    qx1 = cx - cxh - syh
    qy1 = cy - sxh + cyh
    qx2 = cx - cxh + syh
    qy2 = cy - sxh - cyh
    qx3 = cx + cxh + syh
    qy3 = cy + sxh - cyh
    d0x = qx1 - qx0
    d0y = qy1 - qy0
    d1x = qx2 - qx1
    d1y = qy2 - qy1
    zb1 = zc - 0.5 * dz
    zb2 = zc + 0.5 * dz
    vb = dx * dy * dz
    pad = jnp.zeros_like(cx)
    return jnp.stack([cx, cy, cb, sb, hx, hy, zb1, zb2, vb,
                      qx0, qy0, qx1, qy1, qx2, qy2, qx3, qy3,
                      d0x, d0y, d1x, d1y, pad, pad, pad], axis=-1)


def kernel(teacher_boxes, student_boxes, gt_boxes):
    B, N, _ = teacher_boxes.shape
    pred = jnp.concatenate([teacher_boxes, student_boxes], axis=0)
    padbox = jnp.zeros((2 * B, _NPAD - N, 7), jnp.float32).at[:, :, 3:6].set(1.0)
    pred = jnp.concatenate([pred, padbox], axis=1)
    planes = jnp.transpose(pred, (0, 2, 1))
    planes = jnp.concatenate(
        [planes, jnp.zeros((2 * B, 1, _NPAD), jnp.float32)], axis=1)
    planes = planes.reshape(2 * B, 8, _ROWS, _LANES)

    derived = pl.pallas_call(
        _prep_body,
        grid=(_NSETS, _NBLK),
        in_specs=[pl.BlockSpec((1, 8, _RB, _LANES), lambda c, nb: (c, 0, nb, 0))],
        out_specs=pl.BlockSpec((1, _NP, _RB, _LANES), lambda c, nb: (c, 0, nb, 0)),
        out_shape=jax.ShapeDtypeStruct((_NSETS, _NP, _ROWS, _LANES), jnp.float32),
    )(planes)
    derived = derived.reshape(_NSETS, _NP, _NCHUNK, _CHUNK).transpose(0, 2, 1, 3)
    derived = derived.reshape(_NSETS, _NCHUNK, _NP * _CHUNK)

    gtp = _gt_scalars(gt_boxes)
    gtp = jnp.tile(gtp, (2, 1, 1))
    gtb = jnp.broadcast_to(gtp[..., None], (_NSETS, _K, 24, 16))
    gtb = gtb.reshape(_NSETS, _K * 24 * 16)

    sc = functools.partial(
        pl.kernel,
        out_type=jax.ShapeDtypeStruct((_NSETS, _NCHUNK, _CHUNK), jnp.int32),
        mesh=plsc.VectorSubcoreMesh(core_axis_name="c", subcore_axis_name="s"),
        scratch_types=[
            pltpu.VMEM((_NP * _CHUNK,), jnp.float32),
            pltpu.VMEM((_K * 24 * 16,), jnp.float32),
            pltpu.VMEM((_CHUNK,), jnp.int32),
        ],
    )(_sc_assign_body)
    out = sc(derived, gtb)

    out = out.reshape(_NSETS, _NPAD)[:, :N]
    return out[:B], out[B:]


# speedup vs baseline: 1.1955x; 1.1955x over previous
import functools

import jax
import jax.numpy as jnp
from jax import lax
from jax.experimental import pallas as pl
from jax.experimental.pallas import tpu as pltpu
from jax.experimental.pallas import tpu_sc as plsc

_IOU_THRESHOLD = 0.6
_K = 50
_LANES = 128
_RB = 8
_NPAD = 5120
_ROWS = _NPAD // _LANES
_NBLK = _ROWS // _RB
_NSETS = 4
_NCHUNK = 8
_CHUNK = _NPAD // _NCHUNK
_NV = _CHUNK // 16
_NP = 24


def _clip_dt(pu, pv, du, dv, rxu, rxv, hx, hy):
    ta = -(hx + pu) * rxu
    tb = (hx - pu) * rxu
    lo_u = jnp.minimum(ta, tb)
    hi_u = jnp.maximum(ta, tb)
    tc = -(hy + pv) * rxv
    td = (hy - pv) * rxv
    lo_v = jnp.minimum(tc, td)
    hi_v = jnp.maximum(tc, td)
    t0 = jnp.maximum(jnp.maximum(lo_u, lo_v), 0.0)
    t1 = jnp.minimum(jnp.minimum(hi_u, hi_v), 1.0)
    return jnp.maximum(t1 - t0, 0.0)


def _pair_step(p, g, k, best, bki):
    (pax0, pax1, pax2, pax3, pay0, pay1, pay2, pay3,
     e0x, e0y, e1x, e1y, x, y, ca, sa, hxa, hya, za1, za2, va) = p
    (cx, cy, cb, sb, hx, hy, zb1, zb2, vb,
     qx0, qy0, qx1, qy1, qx2, qy2, qx3, qy3, d0x, d0y, d1x, d1y) = g

    us = []
    vs = []
    for pax, pay in ((pax0, pay0), (pax1, pay1), (pax2, pay2), (pax3, pay3)):
        tx = pax - cx
        ty = pay - cy
        us.append(cb * tx + sb * ty)
        vs.append(cb * ty - sb * tx)
    e0u = cb * e0x + sb * e0y
    e0v = cb * e0y - sb * e0x
    e1u = cb * e1x + sb * e1y
    e1v = cb * e1y - sb * e1x
    r0u = 1.0 / e0u
    r0v = 1.0 / e0v
    r1u = 1.0 / e1u
    r1v = 1.0 / e1v
    eds = ((e0u, e0v, r0u, r0v), (e1u, e1v, r1u, r1v),
           (-e0u, -e0v, -r0u, -r0v), (-e1u, -e1v, -r1u, -r1v))
    area2 = jnp.zeros_like(x)
    for j in range(4):
        du, dv, ru, rv = eds[j]
        dt = _clip_dt(us[j], vs[j], du, dv, ru, rv, hx, hy)
        area2 = area2 + dt * (us[j] * dv - vs[j] * du)

    qs = []
    for qxj, qyj in ((qx0, qy0), (qx1, qy1), (qx2, qy2), (qx3, qy3)):
        tx = qxj - x
        ty = qyj - y
        qs.append((ca * tx + sa * ty, ca * ty - sa * tx))
    d0u = ca * d0x + sa * d0y
    d0v = ca * d0y - sa * d0x
    d1u = ca * d1x + sa * d1y
    d1v = ca * d1y - sa * d1x
    s0u = 1.0 / d0u
    s0v = 1.0 / d0v
    s1u = 1.0 / d1u
    s1v = 1.0 / d1v
    bds = ((d0u, d0v, s0u, s0v), (d1u, d1v, s1u, s1v),
           (-d0u, -d0v, -s0u, -s0v), (-d1u, -d1v, -s1u, -s1v))
    dtsum = jnp.zeros_like(x)
    for j in range(4):
        du, dv, ru, rv = bds[j]
        dtsum = dtsum + _clip_dt(qs[j][0], qs[j][1], du, dv, ru, rv, hxa, hya)
    area2 = area2 + dtsum * (2.0 * hx * hy)

    area = jnp.maximum(0.5 * area2, 0.0)
    h = jnp.maximum(jnp.minimum(za2, zb2) - jnp.maximum(za1, zb1), 0.0)
    inter = area * h
    iou = inter / jnp.maximum(va + vb - inter, 1e-6)
    upd = iou > best
    best = jnp.where(upd, iou, best)
    bki = jnp.where(upd, jnp.broadcast_to(k, bki.shape), bki)
    return best, bki


def _prep_body(pred_ref, out_ref):
    x = pred_ref[0, 0]
    y = pred_ref[0, 1]
    z = pred_ref[0, 2]
    dxa = pred_ref[0, 3]
    dya = pred_ref[0, 4]
    dza = pred_ref[0, 5]
    r = pred_ref[0, 6]
    ca = jnp.cos(r)
    sa = jnp.sin(r)
    hxa = 0.5 * dxa
    hya = 0.5 * dya
    cxh = ca * hxa
    sxh = sa * hxa
    cyh = ca * hya
    syh = sa * hya
    out_ref[0, 0] = x + cxh - syh
    out_ref[0, 1] = x - cxh - syh
    out_ref[0, 2] = x - cxh + syh
    out_ref[0, 3] = x + cxh + syh
    out_ref[0, 4] = y + sxh + cyh
    out_ref[0, 5] = y - sxh + cyh
    out_ref[0, 6] = y - sxh - cyh
    out_ref[0, 7] = y + sxh - cyh
    out_ref[0, 8] = -2.0 * cxh
    out_ref[0, 9] = -2.0 * sxh
    out_ref[0, 10] = 2.0 * syh
    out_ref[0, 11] = -2.0 * cyh
    out_ref[0, 12] = x
    out_ref[0, 13] = y
    out_ref[0, 14] = ca
    out_ref[0, 15] = sa
    out_ref[0, 16] = hxa
    out_ref[0, 17] = hya
    out_ref[0, 18] = z - 0.5 * dza
    out_ref[0, 19] = z + 0.5 * dza
    out_ref[0, 20] = dxa * dya * dza
    out_ref[0, 21] = jnp.sqrt(hxa * hxa + hya * hya)
    zero = jnp.zeros_like(x)
    out_ref[0, 22] = zero
    out_ref[0, 23] = zero


def _sc_assign_body(planes_hbm, gt_hbm, out_hbm, pv, gv, ov, fv, bv, kv):
    cid = lax.axis_index("c")
    sid = lax.axis_index("s")
    wid = sid * 2 + cid
    st = wid // _NCHUNK
    ck = lax.rem(wid, _NCHUNK)
    pltpu.sync_copy(planes_hbm.at[st, ck], pv)
    pltpu.sync_copy(gt_hbm.at[st], gv)
    fv[pl.ds(16, 16)] = jnp.zeros((16,), jnp.float32)

    def outer(v, carry_o):
        off = v * 16
        p = tuple(pv[pl.ds(j * _CHUNK + off, 16)] for j in range(21))
        ra = pv[pl.ds(21 * _CHUNK + off, 16)]
        x = p[12]
        y = p[13]
        bv[pl.ds(0, 16)] = jnp.full((16,), -1.0, jnp.float32)
        kv[pl.ds(0, 16)] = jnp.zeros((16,), jnp.int32)

        def inner(k, carry):
            cx = gv[pl.ds(k * 384 + 0 * 16, 16)]
            cy = gv[pl.ds(k * 384 + 1 * 16, 16)]
            rb = gv[pl.ds(k * 384 + 21 * 16, 16)]
            ddx = x - cx
            ddy = y - cy
            rs = ra + rb
            m = jnp.where(ddx * ddx + ddy * ddy < rs * rs, 1.0, 0.0)
            fv[pl.ds(0, 16)] = m
            t1 = fv[pl.ds(0, 16)] + fv[pl.ds(8, 16)]
            fv[pl.ds(0, 16)] = t1
            t2 = fv[pl.ds(0, 16)] + fv[pl.ds(4, 16)]
            s = t2[0] + t2[1] + t2[2] + t2[3]

            @pl.when(s > 0.0)
            def _():
                g = tuple(gv[pl.ds(k * 384 + j * 16, 16)] for j in range(21))
                best, bki = _pair_step(p, g, k, bv[pl.ds(0, 16)],
                                       kv[pl.ds(0, 16)])
                bv[pl.ds(0, 16)] = best
                kv[pl.ds(0, 16)] = bki

            return carry

        lax.fori_loop(0, _K, inner, 0)
        best = bv[pl.ds(0, 16)]
        ov[pl.ds(off, 16)] = jnp.where(best < _IOU_THRESHOLD,
                                       jnp.full((16,), -1, jnp.int32),
                                       kv[pl.ds(0, 16)])
        return carry_o

    lax.fori_loop(0, _NV, outer, 0)
    pltpu.sync_copy(ov, out_hbm.at[st, ck])


def _gt_scalars(gt):
    cx = gt[..., 0]
    cy = gt[..., 1]
    zc = gt[..., 2]
    dx = gt[..., 3]
    dy = gt[..., 4]
    dz = gt[..., 5]
    rr = gt[..., 6]
    cb = jnp.cos(rr)
    sb = jnp.sin(rr)
    hx = 0.5 * dx
    hy = 0.5 * dy
    cxh = cb * hx
    sxh = sb * hx
    cyh = cb * hy
    syh = sb * hy
    qx0 = cx + cxh - syh
    qy0 = cy + sxh + cyh
    qx1 = cx - cxh - syh
    qy1 = cy - sxh + cyh
    qx2 = cx - cxh + syh
    qy2 = cy - sxh - cyh
    qx3 = cx + cxh + syh
    qy3 = cy + sxh - cyh
    d0x = qx1 - qx0
    d0y = qy1 - qy0
    d1x = qx2 - qx1
    d1y = qy2 - qy1
    zb1 = zc - 0.5 * dz
    zb2 = zc + 0.5 * dz
    vb = dx * dy * dz
    rb = jnp.sqrt(hx * hx + hy * hy)
    pad = jnp.zeros_like(cx)
    return jnp.stack([cx, cy, cb, sb, hx, hy, zb1, zb2, vb,
                      qx0, qy0, qx1, qy1, qx2, qy2, qx3, qy3,
                      d0x, d0y, d1x, d1y, rb, pad, pad], axis=-1)


def kernel(teacher_boxes, student_boxes, gt_boxes):
    B, N, _ = teacher_boxes.shape
    pred = jnp.concatenate([teacher_boxes, student_boxes], axis=0)
    padbox = jnp.zeros((2 * B, _NPAD - N, 7), jnp.float32).at[:, :, 3:6].set(1.0)
    pred = jnp.concatenate([pred, padbox], axis=1)
    planes = jnp.transpose(pred, (0, 2, 1))
    planes = jnp.concatenate(
        [planes, jnp.zeros((2 * B, 1, _NPAD), jnp.float32)], axis=1)
    planes = planes.reshape(2 * B, 8, _ROWS, _LANES)

    derived = pl.pallas_call(
        _prep_body,
        grid=(_NSETS, _NBLK),
        in_specs=[pl.BlockSpec((1, 8, _RB, _LANES), lambda c, nb: (c, 0, nb, 0))],
        out_specs=pl.BlockSpec((1, _NP, _RB, _LANES), lambda c, nb: (c, 0, nb, 0)),
        out_shape=jax.ShapeDtypeStruct((_NSETS, _NP, _ROWS, _LANES), jnp.float32),
    )(planes)
    derived = derived.reshape(_NSETS, _NP, _NCHUNK, _CHUNK).transpose(0, 2, 1, 3)
    derived = derived.reshape(_NSETS, _NCHUNK, _NP * _CHUNK)

    gtp = _gt_scalars(gt_boxes)
    gtp = jnp.tile(gtp, (2, 1, 1))
    gtb = jnp.broadcast_to(gtp[..., None], (_NSETS, _K, 24, 16))
    gtb = gtb.reshape(_NSETS, _K * 24 * 16)

    sc = functools.partial(
        pl.kernel,
        out_type=jax.ShapeDtypeStruct((_NSETS, _NCHUNK, _CHUNK), jnp.int32),
        mesh=plsc.VectorSubcoreMesh(core_axis_name="c", subcore_axis_name="s"),
        scratch_types=[
            pltpu.VMEM((_NP * _CHUNK,), jnp.float32),
            pltpu.VMEM((_K * 24 * 16,), jnp.float32),
            pltpu.VMEM((_CHUNK,), jnp.int32),
            pltpu.VMEM((32,), jnp.float32),
            pltpu.VMEM((16,), jnp.float32),
            pltpu.VMEM((16,), jnp.int32),
        ],
    )(_sc_assign_body)
    out = sc(derived, gtb)

    out = out.reshape(_NSETS, _NPAD)[:, :N]
    return out[:B], out[B:]
